# Initial kernel scaffold; baseline (speedup 1.0000x reference)
import jax
import jax.numpy as jnp
from jax.experimental import pallas as pl


def kernel(x, y):
    raise NotImplementedError("write your pallas kernel here")


# speedup vs baseline: 10.1383x; 10.1383x over previous
import math

import jax
import jax.numpy as jnp
from jax.experimental import pallas as pl
from jax.experimental.pallas import tpu as pltpu

N = 4096
DX = 512
DY = 128
TOPK = 10
L = 40
NROWS = 2458
BLK = 128
NBLK = 20
NEG = -1e30

_INVD = tuple(1.0 / math.log2(2.0 + k) if k < TOPK else 0.0 for k in range(L))


def _norm_body(x_ref, y_ref, xn_ref, yn_ref):
    x = x_ref[...]
    nx = jnp.sqrt(jnp.sum(x * x, axis=1, keepdims=True))
    xn_ref[...] = x / jnp.where(nx == 0.0, 1.0, nx)
    y = y_ref[...]
    ny = jnp.sqrt(jnp.sum(y * y, axis=1, keepdims=True))
    yn_ref[...] = y / jnp.where(ny == 0.0, 1.0, ny)


def _main_body(xn_ref, xnT_ref, yn_ref, ynT_ref, out_ref, sx_ref, sxw_ref, syw_ref):
    b = pl.program_id(0)
    row0 = b * BLK
    sx = 5.0 * (jax.lax.dot_general(
        xn_ref[...], xnT_ref[...], (((1,), (0,)), ((), ())),
        preferred_element_type=jnp.float32) + 1.0)
    sy = 5.0 * (jax.lax.dot_general(
        yn_ref[...], ynT_ref[...], (((1,), (0,)), ((), ())),
        preferred_element_type=jnp.float32) + 1.0)
    col = jax.lax.broadcasted_iota(jnp.int32, (BLK, N), 1)
    rid = row0 + jax.lax.broadcasted_iota(jnp.int32, (BLK, N), 0)
    isdiag = col == rid
    sx_ref[...] = sx
    sxw_ref[...] = jnp.where(isdiag, NEG, sx)
    syw_ref[...] = jnp.where(isdiag, NEG, sy)

    l40 = jax.lax.broadcasted_iota(jnp.int32, (BLK, L), 1)

    def step(t, carry):
        y_ss, x_corr, x_ss = carry
        syw = syw_ref[...]
        m = jnp.max(syw, axis=1, keepdims=True)
        idx = jnp.min(jnp.where(syw == m, col, N), axis=1, keepdims=True)
        hot = col == idx
        xg = jnp.sum(jnp.where(hot, sx_ref[...], 0.0), axis=1, keepdims=True)
        syw_ref[...] = jnp.where(hot, NEG, syw)
        y_ss = jnp.where(l40 == t, m, y_ss)
        x_corr = jnp.where(l40 == t, xg, x_corr)
        sxw = sxw_ref[...]
        mx = jnp.max(sxw, axis=1, keepdims=True)
        idxx = jnp.min(jnp.where(sxw == mx, col, N), axis=1, keepdims=True)
        sxw_ref[...] = jnp.where(col == idxx, NEG, sxw)
        x_ss = jnp.where(l40 == t, mx, x_ss)
        return y_ss, x_corr, x_ss

    z = jnp.zeros((BLK, L), jnp.float32)
    y_ss, x_corr, x_ss = jax.lax.fori_loop(0, L, step, (z, z, z))

    l40f = l40.astype(jnp.float32)
    invd = jnp.where(l40 < TOPK, math.log(2.0) / jnp.log(2.0 + l40f), 0.0)
    idcg = jnp.sum((jnp.exp2(x_ss) - 1.0) * invd, axis=1, keepdims=True)
    inv_idcg = 1.0 / idcg
    g = jnp.exp2(x_corr) - 1.0

    acc = jnp.zeros((BLK, L), jnp.float32)
    for k in range(L):
        yk = y_ss[:, k:k + 1]
        xk = x_corr[:, k:k + 1]
        gk = g[:, k:k + 1]
        pd = y_ss - yk
        frac = -1.0 / (1.0 + jnp.exp(pd))
        dd = (g - gk) * (invd - _INVD[k])
        acc = acc + jnp.where((x_corr - xk) > 0.0,
                              frac * jnp.abs(dd) * pd, 0.0)
    rowsum = jnp.sum(acc, axis=1, keepdims=True) * inv_idcg
    rids = row0 + jax.lax.broadcasted_iota(jnp.int32, (BLK, 1), 0)
    mask = (rids < NROWS).astype(jnp.float32)
    loss_blk = jnp.sum(rowsum * mask)

    @pl.when(b == 0)
    def _():
        out_ref[0, 0] = 0.0
    out_ref[0, 0] += loss_blk


def _impl(x, y, interpret=False):
    xn, yn = pl.pallas_call(
        _norm_body,
        out_shape=[jax.ShapeDtypeStruct((N, DX), jnp.float32),
                   jax.ShapeDtypeStruct((N, DY), jnp.float32)],
        interpret=interpret,
    )(x, y)
    xnT = xn.T
    ynT = yn.T
    out = pl.pallas_call(
        _main_body,
        grid=(NBLK,),
        in_specs=[
            pl.BlockSpec((BLK, DX), lambda b: (b, 0)),
            pl.BlockSpec((DX, N), lambda b: (0, 0)),
            pl.BlockSpec((BLK, DY), lambda b: (b, 0)),
            pl.BlockSpec((DY, N), lambda b: (0, 0)),
        ],
        out_specs=pl.BlockSpec((1, 1), lambda b: (0, 0),
                               memory_space=pltpu.SMEM),
        out_shape=jax.ShapeDtypeStruct((1, 1), jnp.float32),
        scratch_shapes=[pltpu.VMEM((BLK, N), jnp.float32)] * 3,
        interpret=interpret,
    )(xn, xnT, yn, ynT)
    return out[0, 0]


def kernel(x, y):
    return _impl(x, y)


# speedup vs baseline: 14.6090x; 1.4410x over previous
import math

import jax
import jax.numpy as jnp
from jax.experimental import pallas as pl
from jax.experimental.pallas import tpu as pltpu

N = 4096
DX = 512
DY = 128
TOPK = 10
L = 40
NROWS = 2458
BLK = 256
NBLK = 10
NEG = -1e30

_INVD = tuple(1.0 / math.log2(2.0 + k) if k < TOPK else 0.0 for k in range(L))


def _norm_body(x_ref, y_ref, xn_ref, yn_ref):
    x = x_ref[...]
    nx = jnp.sqrt(jnp.sum(x * x, axis=1, keepdims=True))
    xn_ref[...] = x / jnp.where(nx == 0.0, 1.0, nx)
    y = y_ref[...]
    ny = jnp.sqrt(jnp.sum(y * y, axis=1, keepdims=True))
    yn_ref[...] = y / jnp.where(ny == 0.0, 1.0, ny)


def _main_body(xn_ref, xnT_ref, yn_ref, ynT_ref, out_ref, sx_ref, sxw_ref, syw_ref):
    b = pl.program_id(0)
    row0 = b * BLK
    sx = 5.0 * (jax.lax.dot_general(
        xn_ref[...], xnT_ref[...], (((1,), (0,)), ((), ())),
        preferred_element_type=jnp.float32) + 1.0)
    sy = 5.0 * (jax.lax.dot_general(
        yn_ref[...], ynT_ref[...], (((1,), (0,)), ((), ())),
        preferred_element_type=jnp.float32) + 1.0)
    col = jax.lax.broadcasted_iota(jnp.int32, (BLK, N), 1)
    rid = row0 + jax.lax.broadcasted_iota(jnp.int32, (BLK, N), 0)
    isdiag = col == rid
    sx_ref[...] = sx
    sxw_ref[...] = jnp.where(isdiag, NEG, sx)
    syw_ref[...] = jnp.where(isdiag, NEG, sy)

    l40 = jax.lax.broadcasted_iota(jnp.int32, (BLK, L), 1)

    def step(t, carry):
        y_ss, x_corr = carry
        syw = syw_ref[...]
        m = jnp.max(syw, axis=1, keepdims=True)
        idx = jnp.min(jnp.where(syw == m, col, N), axis=1, keepdims=True)
        hot = col == idx
        xg = jnp.sum(jnp.where(hot, sx_ref[...], 0.0), axis=1, keepdims=True)
        syw_ref[...] = jnp.where(hot, NEG, syw)
        y_ss = jnp.where(l40 == t, m, y_ss)
        x_corr = jnp.where(l40 == t, xg, x_corr)
        return y_ss, x_corr

    def stepx(t, x_ss):
        sxw = sxw_ref[...]
        mx = jnp.max(sxw, axis=1, keepdims=True)
        idxx = jnp.min(jnp.where(sxw == mx, col, N), axis=1, keepdims=True)
        sxw_ref[...] = jnp.where(col == idxx, NEG, sxw)
        return jnp.where(l10 == t, mx, x_ss)

    l10 = jax.lax.broadcasted_iota(jnp.int32, (BLK, TOPK), 1)
    z = jnp.zeros((BLK, L), jnp.float32)
    y_ss, x_corr = jax.lax.fori_loop(0, L, step, (z, z))
    x_ss = jax.lax.fori_loop(0, TOPK, stepx,
                             jnp.zeros((BLK, TOPK), jnp.float32))

    l40f = l40.astype(jnp.float32)
    invd = jnp.where(l40 < TOPK, math.log(2.0) / jnp.log(2.0 + l40f), 0.0)
    invd10 = math.log(2.0) / jnp.log(2.0 + l10.astype(jnp.float32))
    idcg = jnp.sum((jnp.exp2(x_ss) - 1.0) * invd10, axis=1, keepdims=True)
    inv_idcg = 1.0 / idcg
    g = jnp.exp2(x_corr) - 1.0

    acc = jnp.zeros((BLK, L), jnp.float32)
    for j in range(TOPK):
        yj = y_ss[:, j:j + 1]
        xj = x_corr[:, j:j + 1]
        gj = g[:, j:j + 1]
        pd = yj - y_ss
        frac = -1.0 / (1.0 + jnp.exp(pd))
        dd = (gj - g) * (_INVD[j] - invd)
        acc = acc + jnp.where((xj - x_corr) > 0.0,
                              frac * jnp.abs(dd) * pd, 0.0)
    for k in range(TOPK):
        yk = y_ss[:, k:k + 1]
        xk = x_corr[:, k:k + 1]
        gk = g[:, k:k + 1]
        pd = y_ss - yk
        frac = -1.0 / (1.0 + jnp.exp(pd))
        dd = (g - gk) * (invd - _INVD[k])
        cond = ((x_corr - xk) > 0.0) & (l40 >= TOPK)
        acc = acc + jnp.where(cond, frac * jnp.abs(dd) * pd, 0.0)
    rowsum = jnp.sum(acc, axis=1, keepdims=True) * inv_idcg
    rids = row0 + jax.lax.broadcasted_iota(jnp.int32, (BLK, 1), 0)
    mask = (rids < NROWS).astype(jnp.float32)
    loss_blk = jnp.sum(rowsum * mask)

    @pl.when(b == 0)
    def _():
        out_ref[0, 0] = 0.0
    out_ref[0, 0] += loss_blk


def _impl(x, y, interpret=False):
    xn, yn = pl.pallas_call(
        _norm_body,
        out_shape=[jax.ShapeDtypeStruct((N, DX), jnp.float32),
                   jax.ShapeDtypeStruct((N, DY), jnp.float32)],
        interpret=interpret,
    )(x, y)
    xnT = xn.T
    ynT = yn.T
    out = pl.pallas_call(
        _main_body,
        grid=(NBLK,),
        in_specs=[
            pl.BlockSpec((BLK, DX), lambda b: (b, 0)),
            pl.BlockSpec((DX, N), lambda b: (0, 0)),
            pl.BlockSpec((BLK, DY), lambda b: (b, 0)),
            pl.BlockSpec((DY, N), lambda b: (0, 0)),
        ],
        out_specs=pl.BlockSpec((1, 1), lambda b: (0, 0),
                               memory_space=pltpu.SMEM),
        out_shape=jax.ShapeDtypeStruct((1, 1), jnp.float32),
        scratch_shapes=[pltpu.VMEM((BLK, N), jnp.float32)] * 3,
        interpret=interpret,
    )(xn, xnT, yn, ynT)
    return out[0, 0]


def kernel(x, y):
    return _impl(x, y)
